# baseline (speedup 1.0000x reference)
import jax
import jax.numpy as jnp
from jax.experimental import pallas as pl
from jax.experimental.pallas import tpu as pltpu


_ROWS = 4096
_COLS = 512
_BLK = 256
_GRID = _ROWS // _BLK
_N = float(_ROWS * _COLS)


def _body(x_ref, t_ref, m_ref, h_ref, o_ref, acc_ref):
    i = pl.program_id(0)

    @pl.when(i == 0)
    def _init():
        for k in range(5):
            acc_ref[k] = 0.0

    d = jnp.abs(x_ref[...] - t_ref[...])
    m = m_ref[...] != 0
    h = h_ref[...] != 0
    zero = jnp.zeros_like(d)
    acc_ref[0] += jnp.sum(jnp.where(m, d, zero))
    acc_ref[1] += jnp.sum(m.astype(jnp.float32))
    acc_ref[2] += jnp.sum(jnp.where(h, d, zero))
    acc_ref[3] += jnp.sum(h.astype(jnp.float32))
    acc_ref[4] += jnp.sum(d)

    @pl.when(i == pl.num_programs(0) - 1)
    def _fin():
        s_m, c_m, s_h, c_h, s_t = (acc_ref[0], acc_ref[1], acc_ref[2],
                                   acc_ref[3], acc_ref[4])
        o_ref[0] = (s_m / c_m + s_h / c_h + (s_t - s_h) / (_N - c_h)) / 3.0


def kernel(input, target, masks, hull):
    x = input.reshape(_ROWS, _COLS)
    t = target.reshape(_ROWS, _COLS)
    m = masks.reshape(_ROWS, _COLS).astype(jnp.int8)
    h = hull.reshape(_ROWS, _COLS).astype(jnp.int8)

    spec = pl.BlockSpec((_BLK, _COLS), lambda i: (i, 0))
    out = pl.pallas_call(
        _body,
        grid=(_GRID,),
        in_specs=[spec, spec, spec, spec],
        out_specs=pl.BlockSpec(memory_space=pltpu.SMEM),
        out_shape=jax.ShapeDtypeStruct((1,), jnp.float32),
        scratch_shapes=[pltpu.SMEM((5,), jnp.float32)],
    )(x, t, m, h)
    return out[0]
